# baseline (speedup 1.0000x reference)
import functools

import jax
import jax.numpy as jnp
from jax.experimental import pallas as pl
from jax.experimental.pallas import tpu as pltpu

_BN = 512


def _cd_body(x1_ref, x2t_ref, d1_ref, d2_ref):
    nb = pl.program_id(1)
    x1 = x1_ref[0]
    x2t = x2t_ref[0]
    inner = jax.lax.dot_general(
        x1, x2t, dimension_numbers=(((1,), (0,)), ((), ())),
        preferred_element_type=jnp.float32)
    x1s = jnp.sum(x1 * x1, axis=1, keepdims=True)
    x2s = jnp.sum(x2t * x2t, axis=0, keepdims=True)
    d = (x1s + x2s) - 2.0 * inner
    d1_ref[0] = jnp.min(d, axis=1, keepdims=True)
    part = jnp.min(d, axis=0, keepdims=True)

    @pl.when(nb == 0)
    def _():
        d2_ref[0] = part

    @pl.when(nb > 0)
    def _():
        d2_ref[0] = jnp.minimum(d2_ref[0], part)


@jax.jit
def kernel(xyz1, xyz2):
    B, N, _ = xyz1.shape
    M = xyz2.shape[1]
    x2t = jnp.transpose(xyz2, (0, 2, 1))
    grid = (B, N // _BN)
    d1, d2 = pl.pallas_call(
        _cd_body,
        grid=grid,
        in_specs=[
            pl.BlockSpec((1, _BN, 3), lambda b, i: (b, i, 0)),
            pl.BlockSpec((1, 3, M), lambda b, i: (b, 0, 0)),
        ],
        out_specs=[
            pl.BlockSpec((1, _BN, 1), lambda b, i: (b, i, 0)),
            pl.BlockSpec((1, 1, M), lambda b, i: (b, 0, 0)),
        ],
        out_shape=[
            jax.ShapeDtypeStruct((B, N, 1), jnp.float32),
            jax.ShapeDtypeStruct((B, 1, M), jnp.float32),
        ],
        compiler_params=pltpu.CompilerParams(
            dimension_semantics=("parallel", "arbitrary")),
    )(xyz1, x2t)
    return d1.reshape(B, N), d2.reshape(B, M)
